# baseline (speedup 1.0000x reference)
import jax
import jax.numpy as jnp
from jax import lax
from jax.experimental import pallas as pl

_B = 16384
_V = 1000
_VP = 1024
_D = 128
_BLK = 512


def _tc_body(idx_ref, hi_ref, lo_ref, out_ref):
    i = pl.program_id(0)
    idx = idx_ref[pl.ds(i * _BLK, _BLK)]
    oh = (idx[:, None] == lax.broadcasted_iota(jnp.int32, (_BLK, _VP), 1)).astype(jnp.bfloat16)
    acc = jnp.dot(oh, hi_ref[...], preferred_element_type=jnp.float32)
    acc = acc + jnp.dot(oh, lo_ref[...], preferred_element_type=jnp.float32)
    out_ref[...] = acc


def kernel(noise_levels, table):
    idx = noise_levels.astype(jnp.int32)
    tpad = jnp.zeros((_VP, _D), jnp.float32).at[:_V].set(table)
    hi = tpad.astype(jnp.bfloat16)
    lo = (tpad - hi.astype(jnp.float32)).astype(jnp.bfloat16)
    return pl.pallas_call(
        _tc_body,
        grid=(_B // _BLK,),
        in_specs=[
            pl.BlockSpec((_B,), lambda i: (0,)),
            pl.BlockSpec((_VP, _D), lambda i: (0, 0)),
            pl.BlockSpec((_VP, _D), lambda i: (0, 0)),
        ],
        out_specs=pl.BlockSpec((_BLK, _D), lambda i: (i, 0)),
        out_shape=jax.ShapeDtypeStruct((_B, _D), jnp.float32),
    )(idx, hi, lo)
